# Initial kernel scaffold; baseline (speedup 1.0000x reference)
import jax
import jax.numpy as jnp
from jax.experimental import pallas as pl


def kernel(x, conv1_w, bn1_gamma, bn1_beta, bn1_mean, bn1_var, A_c1_w, A_c1_b, A_bn1_gamma, A_bn1_beta, A_bn1_mean, A_bn1_var, A_c2_w, A_c2_b, A_bn2_gamma, A_bn2_beta, A_bn2_mean, A_bn2_var, A_c3_w, A_c3_b, A_id_w, A_id_b, bnA_gamma, bnA_beta, bnA_mean, bnA_var, B_c1_w, B_c1_b, B_bn1_gamma, B_bn1_beta, B_bn1_mean, B_bn1_var, B_c2_w, B_c2_b, B_bn2_gamma, B_bn2_beta, B_bn2_mean, B_bn2_var, B_c3_w, B_c3_b, B_id_w, B_id_b, bnB_gamma, bnB_beta, bnB_mean, bnB_var, conv13_w, conv13_b, bn13_gamma, bn13_beta, bn13_mean, bn13_var, fc1_w, fc1_b, bn_fc_gamma, bn_fc_beta, bn_fc_mean, bn_fc_var, fc2_w, fc2_b):
    raise NotImplementedError("write your pallas kernel here")


# speedup vs baseline: 1.0021x; 1.0021x over previous
import functools

import jax
import jax.numpy as jnp
from jax.experimental import pallas as pl
from jax.experimental.pallas import tpu as pltpu

_VMEM = 48 * 1024 * 1024
_BUDGET = 6 * 1024 * 1024


def _ru(x, m):
    return (x + m - 1) // m * m


def _conv_body(xph_ref, w_ref, s, kw, th, wo):
    t = pl.program_id(1)
    i = pl.program_id(2)
    h0 = t * th + i // s
    parts = []
    for j in range(kw):
        parts.append(xph_ref[j % s, 0, pl.ds(h0, th), pl.ds(j // s, wo), :])
    lhs = jnp.concatenate(parts, axis=-1) if kw > 1 else parts[0]
    return jnp.dot(lhs.reshape(th * wo, -1), w_ref[0],
                   preferred_element_type=jnp.float32)


def _conv_kernel(xph_ref, w_ref, aff_ref, o_ref, acc_ref, *, s, kw, act):
    i = pl.program_id(2)
    kh = pl.num_programs(2)
    th, wo = o_ref.shape[1], o_ref.shape[2]
    d = _conv_body(xph_ref, w_ref, s, kw, th, wo)

    @pl.when(i == 0)
    def _():
        acc_ref[...] = d

    @pl.when(i > 0)
    def _():
        acc_ref[...] = acc_ref[...] + d

    @pl.when(i == kh - 1)
    def _():
        y = acc_ref[...] * aff_ref[0] + aff_ref[1]
        if act == "relu":
            y = jnp.maximum(y, 0.0)
        o_ref[...] = y.reshape(o_ref.shape).astype(o_ref.dtype)


def _conv_res_kernel(xph_ref, w_ref, aff_ref, res_ref, o_ref, acc_ref, *, s, kw):
    i = pl.program_id(2)
    kh = pl.num_programs(2)
    th, wo = o_ref.shape[1], o_ref.shape[2]
    d = _conv_body(xph_ref, w_ref, s, kw, th, wo)

    @pl.when(i == 0)
    def _():
        acc_ref[...] = d

    @pl.when(i > 0)
    def _():
        acc_ref[...] = acc_ref[...] + d

    @pl.when(i == kh - 1)
    def _():
        y = acc_ref[...] * aff_ref[0] + aff_ref[1]
        r = res_ref[...].astype(jnp.float32).reshape(y.shape)
        y = (y + r) * aff_ref[2] + aff_ref[3]
        o_ref[...] = jnp.maximum(y, 0.0).reshape(o_ref.shape).astype(o_ref.dtype)


def conv2d(x, w, b, stride, pad, scale=None, shift=None, act="none",
           cout_pad=None, res=None, res_scale=None, res_shift=None):
    B, H, W, Cx = x.shape
    KH, KW, Cw, Cout = w.shape
    s = stride
    Ho = (H + 2 * pad - KH) // s + 1
    Wo = (W + 2 * pad - KW) // s + 1
    Hu = Ho + (KH - 1) // s
    Wu = Wo + (KW - 1) // s

    Cinp = _ru(Cx, 128)
    Coutp = Cout if cout_pad is None else cout_pad

    wp = w.astype(jnp.bfloat16)
    if Cinp > Cw:
        wp = jnp.pad(wp, ((0, 0), (0, 0), (0, Cinp - Cw), (0, 0)))
    if Coutp > Cout:
        wp = jnp.pad(wp, ((0, 0), (0, 0), (0, 0), (0, Coutp - Cout)))
    wp = wp.reshape(KH, KW * Cinp, Coutp)

    sc = jnp.ones((Cout,), jnp.float32) if scale is None else scale.astype(jnp.float32)
    sh = jnp.zeros((Cout,), jnp.float32) if shift is None else shift.astype(jnp.float32)
    if b is not None:
        sh = sh + sc * b.astype(jnp.float32)
    if Coutp > Cout:
        sc = jnp.pad(sc, (0, Coutp - Cout))
        sh = jnp.pad(sh, (0, Coutp - Cout))
    if res is not None:
        aff = jnp.stack([sc, sh, res_scale.astype(jnp.float32),
                         res_shift.astype(jnp.float32)], axis=0)
    else:
        aff = jnp.stack([sc, sh], axis=0)

    xp = x.astype(jnp.bfloat16)
    if pad or Cinp > Cx:
        xp = jnp.pad(xp, ((0, 0), (pad, pad), (pad, pad), (0, Cinp - Cx)))
    if s == 1:
        xph = xp[None]
    else:
        phases = []
        for a in range(s):
            for bb in range(s):
                ph = xp[:, a::s, bb::s, :][:, :Hu, :Wu, :]
                dh, dw = Hu - ph.shape[1], Wu - ph.shape[2]
                if dh or dw:
                    ph = jnp.pad(ph, ((0, 0), (0, dh), (0, dw), (0, 0)))
                phases.append(ph)
        xph = jnp.stack(phases, axis=0)

    per_row = Wo * (4 * Coutp + 2 * KW * Cinp + 2 * Coutp)
    th = 1
    for d in range(Ho, 0, -1):
        if Ho % d == 0 and d * per_row <= _BUDGET:
            th = d
            break
    n_h = Ho // th

    in_specs = [
        pl.BlockSpec((s, 1, Hu, Wu, Cinp),
                     lambda bi, t, i: ((i % s) if s > 1 else 0, bi, 0, 0, 0)),
        pl.BlockSpec((1, KW * Cinp, Coutp), lambda bi, t, i: (i, 0, 0)),
        pl.BlockSpec((aff.shape[0], Coutp), lambda bi, t, i: (0, 0)),
    ]
    ops = [xph, wp, aff]
    if res is not None:
        in_specs.append(pl.BlockSpec((1, th, Wo, Coutp),
                                     lambda bi, t, i: (bi, t, 0, 0)))
        ops.append(res)
        body = functools.partial(_conv_res_kernel, s=s, kw=KW)
    else:
        body = functools.partial(_conv_kernel, s=s, kw=KW, act=act)

    return pl.pallas_call(
        body,
        out_shape=jax.ShapeDtypeStruct((B, Ho, Wo, Coutp), jnp.bfloat16),
        grid=(B, n_h, KH),
        in_specs=in_specs,
        out_specs=pl.BlockSpec((1, th, Wo, Coutp), lambda bi, t, i: (bi, t, 0, 0)),
        scratch_shapes=[pltpu.VMEM((th * Wo, Coutp), jnp.float32)],
        compiler_params=pltpu.CompilerParams(
            dimension_semantics=("parallel", "parallel", "arbitrary"),
            vmem_limit_bytes=_VMEM),
    )(*ops)


def _stem_mm_kernel(x_ref, w_ref, aff_ref, o_ref):
    d = jnp.dot(x_ref[...], w_ref[...], preferred_element_type=jnp.float32)
    y = jnp.maximum(d * aff_ref[0] + aff_ref[1], 0.0)
    o_ref[...] = y.astype(o_ref.dtype)


def stem_matmul(x, w, scale, shift):
    M, K = x.shape
    Kw, N = w.shape
    Kp, Np = _ru(K, 128), _ru(N, 128)
    tm = min(_ru(M, 8), 2048)
    Mp = _ru(M, tm)
    xp = jnp.pad(x.astype(jnp.bfloat16), ((0, Mp - M), (0, Kp - K)))
    wp = jnp.pad(w.astype(jnp.bfloat16), ((0, Kp - K), (0, Np - N)))
    aff = jnp.stack([jnp.pad(scale.astype(jnp.float32), (0, Np - N)),
                     jnp.pad(shift.astype(jnp.float32), (0, Np - N))], axis=0)
    out = pl.pallas_call(
        _stem_mm_kernel,
        out_shape=jax.ShapeDtypeStruct((Mp, Np), jnp.bfloat16),
        grid=(Mp // tm,),
        in_specs=[pl.BlockSpec((tm, Kp), lambda i: (i, 0)),
                  pl.BlockSpec((Kp, Np), lambda i: (0, 0)),
                  pl.BlockSpec((2, Np), lambda i: (0, 0))],
        out_specs=pl.BlockSpec((tm, Np), lambda i: (i, 0)),
        compiler_params=pltpu.CompilerParams(
            dimension_semantics=("parallel",),
            vmem_limit_bytes=_VMEM),
    )(xp, wp, aff)
    return out[:M]


def _maxpool_kernel(ph_ref, o_ref, *, k, s):
    ho, wo = o_ref.shape[1], o_ref.shape[2]
    acc = None
    for i in range(k):
        for j in range(k):
            p = (i % s) * s + (j % s)
            v = ph_ref[p, 0, pl.ds(i // s, ho), pl.ds(j // s, wo), :]
            acc = v if acc is None else jnp.maximum(acc, v)
    o_ref[...] = acc[None]


def maxpool2d(x, k, s, pad=0):
    B, H, W, C = x.shape
    Ho = (H + 2 * pad - k) // s + 1
    Wo = (W + 2 * pad - k) // s + 1
    Hu = Ho + (k - 1) // s
    Wu = Wo + (k - 1) // s
    xp = x
    if pad:
        xp = jnp.pad(x, ((0, 0), (pad, pad), (pad, pad), (0, 0)),
                     constant_values=-jnp.inf)
    if s == 1:
        xph = xp[None]
    else:
        phases = []
        for a in range(s):
            for bb in range(s):
                ph = xp[:, a::s, bb::s, :][:, :Hu, :Wu, :]
                dh, dw = Hu - ph.shape[1], Wu - ph.shape[2]
                if dh or dw:
                    ph = jnp.pad(ph, ((0, 0), (0, dh), (0, dw), (0, 0)),
                                 constant_values=-jnp.inf)
                phases.append(ph)
        xph = jnp.stack(phases, axis=0)
    nph = xph.shape[0]
    return pl.pallas_call(
        functools.partial(_maxpool_kernel, k=k, s=s),
        out_shape=jax.ShapeDtypeStruct((B, Ho, Wo, C), x.dtype),
        grid=(B,),
        in_specs=[pl.BlockSpec((nph, 1, Hu, Wu, C), lambda bi: (0, bi, 0, 0, 0))],
        out_specs=pl.BlockSpec((1, Ho, Wo, C), lambda bi: (bi, 0, 0, 0)),
        compiler_params=pltpu.CompilerParams(
            dimension_semantics=("parallel",),
            vmem_limit_bytes=_VMEM),
    )(xph)


def _head_kernel(x_ref, w1_ref, a1_ref, w2_ref, b2_ref, o_ref):
    h = jnp.dot(x_ref[...], w1_ref[...], preferred_element_type=jnp.float32)
    h = jnp.maximum(h * a1_ref[0] + a1_ref[1], 0.0).astype(jnp.bfloat16)
    y = jnp.dot(h, w2_ref[...], preferred_element_type=jnp.float32)
    o_ref[...] = jax.nn.sigmoid(y + b2_ref[0])


def mlp_head(x, w1, b1, sc1, sh1, w2, b2):
    M, K = x.shape
    N1 = w1.shape[1]
    N2 = w2.shape[1]
    N2p = _ru(N2, 128)
    sh1f = sh1.astype(jnp.float32) + sc1.astype(jnp.float32) * b1.astype(jnp.float32)
    a1 = jnp.stack([sc1.astype(jnp.float32), sh1f], axis=0)
    w2p = jnp.pad(w2.astype(jnp.bfloat16), ((0, 0), (0, N2p - N2)))
    b2p = jnp.pad(b2.astype(jnp.float32), (0, N2p - N2))[None]
    out = pl.pallas_call(
        _head_kernel,
        out_shape=jax.ShapeDtypeStruct((M, N2p), jnp.float32),
        compiler_params=pltpu.CompilerParams(vmem_limit_bytes=_VMEM),
    )(x.astype(jnp.bfloat16), w1.astype(jnp.bfloat16), a1, w2p, b2p)
    return out[:, :N2]


def _bn_ss(gamma, beta, mean, var, eps=1e-5):
    s = gamma / jnp.sqrt(var + eps)
    return s, beta - mean * s


def _im2col(x, kh, kw, stride, pad):
    B, H, W, C = x.shape
    Ho = (H + 2 * pad - kh) // stride + 1
    Wo = (W + 2 * pad - kw) // stride + 1
    xp = jnp.pad(x, ((0, 0), (pad, pad), (pad, pad), (0, 0)))
    cols = [xp[:, i:i + stride * Ho:stride, j:j + stride * Wo:stride, :]
            for i in range(kh) for j in range(kw)]
    cols = jnp.stack(cols, axis=3)
    return cols.reshape(B * Ho * Wo, kh * kw * C), (B, Ho, Wo)


def kernel(x, conv1_w, bn1_gamma, bn1_beta, bn1_mean, bn1_var, A_c1_w, A_c1_b, A_bn1_gamma, A_bn1_beta, A_bn1_mean, A_bn1_var, A_c2_w, A_c2_b, A_bn2_gamma, A_bn2_beta, A_bn2_mean, A_bn2_var, A_c3_w, A_c3_b, A_id_w, A_id_b, bnA_gamma, bnA_beta, bnA_mean, bnA_var, B_c1_w, B_c1_b, B_bn1_gamma, B_bn1_beta, B_bn1_mean, B_bn1_var, B_c2_w, B_c2_b, B_bn2_gamma, B_bn2_beta, B_bn2_mean, B_bn2_var, B_c3_w, B_c3_b, B_id_w, B_id_b, bnB_gamma, bnB_beta, bnB_mean, bnB_var, conv13_w, conv13_b, bn13_gamma, bn13_beta, bn13_mean, bn13_var, fc1_w, fc1_b, bn_fc_gamma, bn_fc_beta, bn_fc_mean, bn_fc_var, fc2_w, fc2_b):
    xh = jnp.transpose(x, (0, 2, 3, 1)).astype(jnp.bfloat16)

    sc, sh = _bn_ss(bn1_gamma, bn1_beta, bn1_mean, bn1_var)
    cols, (B, Ho, Wo) = _im2col(xh, 7, 7, 2, 3)
    h = stem_matmul(cols, conv1_w.reshape(49, 64), sc, sh)
    h = h.reshape(B, Ho, Wo, -1)
    h = maxpool2d(h, 3, 2, 1)

    idn = conv2d(h, A_id_w, A_id_b, 2, 2)
    sc, sh = _bn_ss(A_bn1_gamma, A_bn1_beta, A_bn1_mean, A_bn1_var)
    y = conv2d(h, A_c1_w, A_c1_b, 1, 2, sc, sh, "relu", cout_pad=256)
    sc, sh = _bn_ss(A_bn2_gamma, A_bn2_beta, A_bn2_mean, A_bn2_var)
    y = conv2d(y, A_c2_w, A_c2_b, 1, 2, sc, sh, "relu")
    scA, shA = _bn_ss(bnA_gamma, bnA_beta, bnA_mean, bnA_var)
    h = conv2d(y, A_c3_w, A_c3_b, 2, 2, res=idn, res_scale=scA, res_shift=shA)
    h = maxpool2d(h, 3, 2, 1)

    idn = conv2d(h, B_id_w, B_id_b, 3, 3)
    sc, sh = _bn_ss(B_bn1_gamma, B_bn1_beta, B_bn1_mean, B_bn1_var)
    y = conv2d(h, B_c1_w, B_c1_b, 1, 3, sc, sh, "relu")
    sc, sh = _bn_ss(B_bn2_gamma, B_bn2_beta, B_bn2_mean, B_bn2_var)
    y = conv2d(y, B_c2_w, B_c2_b, 1, 3, sc, sh, "relu")
    scB, shB = _bn_ss(bnB_gamma, bnB_beta, bnB_mean, bnB_var)
    h = conv2d(y, B_c3_w, B_c3_b, 3, 3, res=idn, res_scale=scB, res_shift=shB)
    h = maxpool2d(h, 3, 1, 0)

    sc, sh = _bn_ss(bn13_gamma, bn13_beta, bn13_mean, bn13_var)
    h = conv2d(h, conv13_w, conv13_b, 1, 0, sc, sh, "relu")
    h = maxpool2d(h, 2, 2, 0)

    flat = h.reshape(h.shape[0], h.shape[3])
    sc, sh = _bn_ss(bn_fc_gamma, bn_fc_beta, bn_fc_mean, bn_fc_var)
    return mlp_head(flat, fc1_w, fc1_b, sc, sh, fc2_w, fc2_b)
